# baseline (speedup 1.0000x reference)
import functools

import jax
import jax.numpy as jnp
from jax import lax
from jax.experimental import pallas as pl
from jax.experimental.pallas import tpu as pltpu
from jax.experimental.pallas import tpu_sc as plsc

D = 1024
C = 16
NW = 32
LANES = 16


def _sc_add_pos_enc(x2d, pos, table):
    rows = x2d.shape[0]
    nchunks = rows // C
    mesh = plsc.VectorSubcoreMesh(core_axis_name="c", subcore_axis_name="s")

    @functools.partial(
        pl.kernel,
        out_type=jax.ShapeDtypeStruct((rows, D), jnp.float32),
        mesh=mesh,
        scratch_types=[
            pltpu.VMEM((C,), jnp.int32),
            pltpu.VMEM((C, D), jnp.float32),
            pltpu.VMEM((C, D), jnp.float32),
            pltpu.SemaphoreType.DMA,
            pltpu.SemaphoreType.DMA,
        ],
    )
    def run(x_hbm, pos_hbm, tab_hbm, out_hbm, idx_v, xbuf, rowbuf, sem_x, sem_g):
        cid = lax.axis_index("c")
        sid = lax.axis_index("s")
        wid = sid * 2 + cid
        my_n = (nchunks - 1 - wid) // NW + 1

        def chunk_body(i, carry):
            base = (wid + i * NW) * C
            pltpu.sync_copy(pos_hbm.at[pl.ds(base, C)], idx_v)
            cpx = pltpu.async_copy(x_hbm.at[pl.ds(base, C)], xbuf, sem_x)
            cpg = pltpu.async_copy(tab_hbm.at[idx_v], rowbuf, sem_g)
            cpx.wait()
            cpg.wait()

            def row_body(r, rcarry):
                for j in range(D // LANES):
                    sl = pl.ds(j * LANES, LANES)
                    plsc.addupdate(xbuf.at[r, sl], rowbuf[r, sl])
                return rcarry

            lax.fori_loop(0, C, row_body, 0)
            pltpu.sync_copy(xbuf, out_hbm.at[pl.ds(base, C)])
            return carry

        lax.fori_loop(0, my_n, chunk_body, 0)

    return run(x2d, pos, table)


def kernel(x, positions, pos_table):
    b, s, d = x.shape
    x2d = x.reshape(b * s, d)
    pos = positions.reshape(b * s).astype(jnp.int32)
    out = _sc_add_pos_enc(x2d, pos, pos_table)
    return out.reshape(b, s, d)


# speedup vs baseline: 1.1499x; 1.1499x over previous
import functools

import jax
import jax.numpy as jnp
from jax import lax
from jax.experimental import pallas as pl
from jax.experimental.pallas import tpu as pltpu
from jax.experimental.pallas import tpu_sc as plsc

D = 1024
C = 16
NW = 32
LANES = 16
NB = 3


def _sc_add_pos_enc(x2d, pos, table):
    rows = x2d.shape[0]
    nchunks = rows // C
    ni = -(-nchunks // NW)
    nblocks = -(-ni // NB)
    mesh = plsc.VectorSubcoreMesh(core_axis_name="c", subcore_axis_name="s")

    @functools.partial(
        pl.kernel,
        out_type=jax.ShapeDtypeStruct((rows, D), jnp.float32),
        mesh=mesh,
        scratch_types=[
            pltpu.VMEM((rows,), jnp.int32),
            [pltpu.VMEM((C, D), jnp.float32)] * NB,
            [pltpu.VMEM((C, D), jnp.float32)] * NB,
            [pltpu.SemaphoreType.DMA] * NB,
            [pltpu.SemaphoreType.DMA] * NB,
            [pltpu.SemaphoreType.DMA] * NB,
        ],
    )
    def run(x_hbm, pos_hbm, tab_hbm, out_hbm,
            pos_v, xbufs, rowbufs, sems_x, sems_g, sems_o):
        cid = lax.axis_index("c")
        sid = lax.axis_index("s")
        wid = sid * 2 + cid
        my_n = (nchunks - 1 - wid) // NW + 1

        pltpu.sync_copy(pos_hbm, pos_v)

        def issue(i, b):
            base = (wid + i * NW) * C
            pltpu.async_copy(x_hbm.at[pl.ds(base, C)], xbufs[b], sems_x[b])
            pltpu.async_copy(tab_hbm.at[pos_v.at[pl.ds(base, C)]],
                             rowbufs[b], sems_g[b])

        def finish(i, b):
            base = (wid + i * NW) * C
            pltpu.make_async_copy(x_hbm.at[pl.ds(base, C)], xbufs[b],
                                  sems_x[b]).wait()
            pltpu.make_async_copy(tab_hbm.at[pl.ds(0, C)], rowbufs[b],
                                  sems_g[b]).wait()

            def row_body(r, rcarry):
                for j in range(D // LANES):
                    sl = pl.ds(j * LANES, LANES)
                    plsc.addupdate(xbufs[b].at[r, sl], rowbufs[b][r, sl])
                return rcarry

            lax.fori_loop(0, C, row_body, 0)
            pltpu.async_copy(xbufs[b], out_hbm.at[pl.ds(base, C)], sems_o[b])

        for b in range(NB):
            issue(b, b)

        def block_body(blk, carry):
            i0 = blk * NB
            for b in range(NB):
                @pl.when(i0 + b < my_n)
                def _():
                    finish(i0 + b, b)
            for b in range(NB):
                nxt = i0 + NB + b

                @pl.when(nxt < my_n)
                def _():
                    pltpu.make_async_copy(
                        xbufs[b], out_hbm.at[pl.ds(0, C)], sems_o[b]).wait()
                    issue(nxt, b)
            return carry

        lax.fori_loop(0, nblocks, block_body, 0)

        for b in range(NB):
            pltpu.make_async_copy(
                xbufs[b], out_hbm.at[pl.ds(0, C)], sems_o[b]).wait()

    return run(x2d, pos, table)


def kernel(x, positions, pos_table):
    b, s, d = x.shape
    x2d = x.reshape(b * s, d)
    pos = positions.reshape(b * s).astype(jnp.int32)
    out = _sc_add_pos_enc(x2d, pos, pos_table)
    return out.reshape(b, s, d)
